# baseline (speedup 1.0000x reference)
import jax
import jax.numpy as jnp
from jax.experimental import pallas as pl
from jax.experimental.pallas import tpu as pltpu


def _decode_x_body(x_ref, k_ref, o_ref):
    k = k_ref[...]
    m = jnp.dot(k, k.T, preferred_element_type=jnp.float32)
    o_ref[...] = jnp.dot(x_ref[...], m, preferred_element_type=jnp.float32)


def _matmul_body(xm_ref, w_ref, b_ref, o_ref):
    acc = jax.lax.dot_general(
        xm_ref[...],
        w_ref[...],
        dimension_numbers=(((1,), (1,)), ((), ())),
        preferred_element_type=jnp.float32,
    )
    o_ref[...] = acc + b_ref[...]


def kernel(input_encodings, weight_encodings, G, bias):
    B, I, D = input_encodings.shape
    O = weight_encodings.shape[0]
    K = I * D

    LANE = 128
    reps = LANE // D
    x2 = input_encodings.reshape(B * K // LANE, LANE)
    w2 = weight_encodings.reshape(O, K)
    bias2 = bias.reshape(1, O)
    kmat = jnp.kron(jnp.eye(reps, dtype=G.dtype), G)

    xm = pl.pallas_call(
        _decode_x_body,
        out_shape=jax.ShapeDtypeStruct(x2.shape, jnp.float32),
    )(x2, kmat)
    xm2 = xm.reshape(B, K)

    BO = 256
    out = pl.pallas_call(
        _matmul_body,
        grid=(O // BO,),
        in_specs=[
            pl.BlockSpec((B, K), lambda o: (0, 0)),
            pl.BlockSpec((BO, K), lambda o: (o, 0)),
            pl.BlockSpec((1, BO), lambda o: (0, o)),
        ],
        out_specs=pl.BlockSpec((B, BO), lambda o: (0, o)),
        out_shape=jax.ShapeDtypeStruct((B, O), jnp.float32),
        compiler_params=pltpu.CompilerParams(
            dimension_semantics=("parallel",)
        ),
    )(xm2, w2, bias2)
    return out


# speedup vs baseline: 5.7843x; 5.7843x over previous
import functools

import jax
import jax.numpy as jnp
from jax.experimental import pallas as pl
from jax.experimental.pallas import tpu as pltpu


def _fused_body(nbuf, jb, nj, ib, xp_ref, kb_ref, sel_ref, b_ref, w_hbm,
                o_ref, wbuf, xms, sems):
    nchunks = nj // jb

    def copyj(c, jj, slot):
        return pltpu.make_async_copy(
            w_hbm.at[:, c * jb + jj, :],
            wbuf.at[slot, jj],
            sems.at[slot, jj],
        )

    for s in range(min(nbuf, nchunks)):
        for jj in range(jb):
            copyj(s, jj, s).start()

    y = jax.lax.dot_general(
        kb_ref[...], xp_ref[...],
        dimension_numbers=(((0,), (0,)), ((), ())),
        preferred_element_type=jnp.float32,
    )
    d_dim = nj // ib
    for d in range(d_dim):
        xm_d = jnp.dot(sel_ref[d], y, preferred_element_type=jnp.float32)
        for blk in range(ib):
            xms[blk * d_dim + d] = xm_d[:, blk * 128:(blk + 1) * 128].astype(
                jnp.bfloat16
            )

    for c in range(nchunks):
        slot = c % nbuf
        for jj in range(jb):
            copyj(c, jj, slot).wait()
        acc = None
        for jj in range(jb):
            part = jax.lax.dot_general(
                xms[c * jb + jj],
                wbuf[slot, jj].astype(jnp.bfloat16),
                dimension_numbers=(((1,), (1,)), ((), ())),
                preferred_element_type=jnp.float32,
            )
            acc = part if acc is None else acc + part
        if c == 0:
            o_ref[...] = acc + b_ref[...]
        else:
            o_ref[...] = o_ref[...] + acc
        if c + nbuf < nchunks:
            for jj in range(jb):
                copyj(c + nbuf, jj, slot).start()


def kernel(input_encodings, weight_encodings, G, bias):
    B, I, D = input_encodings.shape
    O = weight_encodings.shape[0]
    NJ = I * D // 128
    IB = I // 128

    xp = jnp.transpose(input_encodings, (0, 2, 1)).reshape(B * D, I)
    v4 = (
        weight_encodings.reshape(O, IB, 128, D)
        .transpose(0, 1, 3, 2)
        .reshape(O, NJ, 128)
    )
    bias2 = bias.reshape(1, O)
    eye = jnp.eye(B, dtype=G.dtype)
    kb = jnp.kron(eye, G)
    sel = jnp.stack([jnp.kron(eye, G[d:d + 1, :]) for d in range(D)])

    NBUF = 6
    JB = 8
    out = pl.pallas_call(
        functools.partial(_fused_body, NBUF, JB, NJ, IB),
        in_specs=[
            pl.BlockSpec((B * D, I), lambda: (0, 0)),
            pl.BlockSpec((B * D, B * D), lambda: (0, 0)),
            pl.BlockSpec((D, B, B * D), lambda: (0, 0, 0)),
            pl.BlockSpec((1, O), lambda: (0, 0)),
            pl.BlockSpec(memory_space=pl.ANY),
        ],
        out_specs=pl.BlockSpec((B, O), lambda: (0, 0)),
        out_shape=jax.ShapeDtypeStruct((B, O), jnp.float32),
        scratch_shapes=[
            pltpu.VMEM((NBUF, JB, O, 128), jnp.float32),
            pltpu.VMEM((NJ, B, 128), jnp.bfloat16),
            pltpu.SemaphoreType.DMA((NBUF, JB)),
        ],
    )(xp, kb, sel, bias2, v4)
    return out
